# baseline (speedup 1.0000x reference)
import functools

import jax
import jax.numpy as jnp
from jax.experimental import pallas as pl
from jax.experimental.pallas import tpu as pltpu

_NBUF = 4


def _moe_body(uexp_ref, starts_ref, cnts_ref, pairs_ref, wsort_ref, d_ref,
              x_ref, w1_any, w3_any, w2_any, out_ref,
              w1b, w3b, w2b, sems, *, top_k):
    d = d_ref[0]
    out_ref[...] = jnp.zeros_like(out_ref)

    def _copies(j, slot):
        e = uexp_ref[j]
        return (
            pltpu.make_async_copy(w1_any.at[e], w1b.at[slot], sems.at[slot, 0]),
            pltpu.make_async_copy(w3_any.at[e], w3b.at[slot], sems.at[slot, 1]),
            pltpu.make_async_copy(w2_any.at[e], w2b.at[slot], sems.at[slot, 2]),
        )

    for jj in range(_NBUF):
        @pl.when(jj < d)
        def _start():
            for c in _copies(jj, jj):
                c.start()

    def seg_body(j, carry):
        slot = jax.lax.rem(j, _NBUF)
        for c in _copies(j, slot):
            c.wait()

        start = starts_ref[j]
        cnt = cnts_ref[j]

        def row_body(r, carry2):
            p = pairs_ref[r]
            t = p // top_k
            xrow = x_ref[pl.ds(t, 1), :]
            g = jnp.dot(xrow, w1b[slot], preferred_element_type=jnp.float32)
            u = jnp.dot(xrow, w3b[slot], preferred_element_type=jnp.float32)
            h = (g * jax.nn.sigmoid(g)) * u
            o = jnp.dot(h, w2b[slot], preferred_element_type=jnp.float32)
            out_ref[pl.ds(t, 1), :] += wsort_ref[r] * o
            return carry2

        jax.lax.fori_loop(start, start + cnt, row_body, 0)

        @pl.when(j + _NBUF < d)
        def _next():
            for c in _copies(j + _NBUF, slot):
                c.start()
        return carry

    jax.lax.fori_loop(0, d, seg_body, 0)


def _route(eids, wvals):
    P = eids.shape[0]
    ii = jnp.arange(P, dtype=jnp.int32)
    lt = eids[None, :] < eids[:, None]
    tie = (eids[None, :] == eids[:, None]) & (ii[None, :] < ii[:, None])
    rank = (lt | tie).sum(axis=1).astype(jnp.int32)
    onehot = rank[None, :] == ii[:, None]
    sorted_eids = jnp.where(onehot, eids[None, :], 0).sum(axis=1)
    order = jnp.where(onehot, ii[None, :], 0).sum(axis=1).astype(jnp.int32)
    sorted_w = wvals[order]

    new = jnp.concatenate([jnp.ones((1,), jnp.int32),
                           (sorted_eids[1:] != sorted_eids[:-1]).astype(jnp.int32)])
    segid = jnp.cumsum(new) - 1
    d = new.sum().astype(jnp.int32)
    seg_onehot = segid[None, :] == ii[:, None]
    is_first = seg_onehot & (new[None, :] == 1)
    uexp = jnp.where(is_first, sorted_eids[None, :], 0).sum(axis=1).astype(jnp.int32)
    starts = jnp.where(is_first, ii[None, :], 0).sum(axis=1).astype(jnp.int32)
    cnts = seg_onehot.sum(axis=1).astype(jnp.int32)
    return uexp, starts, cnts, order, sorted_w, d


def kernel(x, expert_indices, expert_weights, w1_stacked, w2_stacked, w3_stacked):
    B, H = x.shape
    K = expert_indices.shape[1]
    E, _, I = w1_stacked.shape
    P = B * K

    eids = expert_indices.reshape(P).astype(jnp.int32)
    uexp, starts, cnts, order, sorted_w, d = _route(eids, expert_weights.reshape(P))
    darr = d.reshape(1)

    grid_spec = pltpu.PrefetchScalarGridSpec(
        num_scalar_prefetch=6,
        grid=(1,),
        in_specs=[
            pl.BlockSpec((B, H), lambda i, *_: (0, 0)),
            pl.BlockSpec(memory_space=pl.ANY),
            pl.BlockSpec(memory_space=pl.ANY),
            pl.BlockSpec(memory_space=pl.ANY),
        ],
        out_specs=pl.BlockSpec((B, H), lambda i, *_: (0, 0)),
        scratch_shapes=[
            pltpu.VMEM((_NBUF, H, I), jnp.float32),
            pltpu.VMEM((_NBUF, H, I), jnp.float32),
            pltpu.VMEM((_NBUF, I, H), jnp.float32),
            pltpu.SemaphoreType.DMA((_NBUF, 3)),
        ],
    )
    fn = pl.pallas_call(
        functools.partial(_moe_body, top_k=K),
        grid_spec=grid_spec,
        out_shape=jax.ShapeDtypeStruct((B, H), jnp.float32),
    )
    return fn(uexp, starts, cnts, order, sorted_w, darr,
              x, w1_stacked, w3_stacked, w2_stacked)


# speedup vs baseline: 1.0094x; 1.0094x over previous
import functools

import jax
import jax.numpy as jnp
from jax.experimental import pallas as pl
from jax.experimental.pallas import tpu as pltpu

_NBUF = 3


def _moe_body(eids_ref, ew_ref, x_ref, w1_any, w3_any, w2_any, out_ref,
              w1b, w3b, w2b, cnt, base, uexp, starts, scnt, srows, sems,
              *, top_k, n_experts):
    P = eids_ref.shape[0]
    out_ref[...] = jnp.zeros_like(out_ref)
    zero = jnp.int32(0)

    def clear_body(e, c):
        cnt[e] = zero
        return c
    jax.lax.fori_loop(0, n_experts, clear_body, zero)

    def count_body(i, c):
        cnt[eids_ref[i]] += 1
        return c
    jax.lax.fori_loop(0, P, count_body, zero)

    def seg_scan(e, carry):
        d, pos = carry
        c = cnt[e]

        @pl.when(c > 0)
        def _():
            uexp[d] = e
            starts[d] = pos
            scnt[d] = c
        base[e] = pos
        return jnp.where(c > 0, d + 1, d), pos + c

    d, _ = jax.lax.fori_loop(0, n_experts, seg_scan, (zero, zero))

    def scatter_body(i, c):
        e = eids_ref[i]
        b = base[e]
        srows[b] = i
        base[e] = b + 1
        return c
    jax.lax.fori_loop(0, P, scatter_body, zero)

    def _copies(j, slot):
        e = uexp[j]
        return (
            pltpu.make_async_copy(w1_any.at[e], w1b.at[slot], sems.at[slot, 0]),
            pltpu.make_async_copy(w3_any.at[e], w3b.at[slot], sems.at[slot, 1]),
            pltpu.make_async_copy(w2_any.at[e], w2b.at[slot], sems.at[slot, 2]),
        )

    for jj in range(_NBUF):
        @pl.when(jj < d)
        def _start():
            for c in _copies(jj, jj):
                c.start()

    def seg_body(j, carry):
        slot = jax.lax.rem(j, _NBUF)
        for c in _copies(j, slot):
            c.wait()

        start = starts[j]

        def row_body(r, carry2):
            p = srows[r]
            t = p // top_k
            xrow = x_ref[pl.ds(t, 1), :]
            g = jnp.dot(xrow, w1b[slot], preferred_element_type=jnp.float32)
            u = jnp.dot(xrow, w3b[slot], preferred_element_type=jnp.float32)
            h = (g * jax.nn.sigmoid(g)) * u
            o = jnp.dot(h, w2b[slot], preferred_element_type=jnp.float32)
            out_ref[pl.ds(t, 1), :] += ew_ref[p] * o
            return carry2

        jax.lax.fori_loop(start, start + scnt[j], row_body, zero)

        @pl.when(j + _NBUF < d)
        def _next():
            for c in _copies(j + _NBUF, slot):
                c.start()
        return carry

    jax.lax.fori_loop(0, d, seg_body, zero)


def kernel(x, expert_indices, expert_weights, w1_stacked, w2_stacked, w3_stacked):
    B, H = x.shape
    K = expert_indices.shape[1]
    E, _, I = w1_stacked.shape
    P = B * K

    eids = expert_indices.reshape(P).astype(jnp.int32)
    ew = expert_weights.reshape(P)

    grid_spec = pltpu.PrefetchScalarGridSpec(
        num_scalar_prefetch=2,
        grid=(1,),
        in_specs=[
            pl.BlockSpec((B, H), lambda i, *_: (0, 0)),
            pl.BlockSpec(memory_space=pl.ANY),
            pl.BlockSpec(memory_space=pl.ANY),
            pl.BlockSpec(memory_space=pl.ANY),
        ],
        out_specs=pl.BlockSpec((B, H), lambda i, *_: (0, 0)),
        scratch_shapes=[
            pltpu.VMEM((_NBUF, H, I), jnp.float32),
            pltpu.VMEM((_NBUF, H, I), jnp.float32),
            pltpu.VMEM((_NBUF, I, H), jnp.float32),
            pltpu.SMEM((E,), jnp.int32),
            pltpu.SMEM((E,), jnp.int32),
            pltpu.SMEM((P,), jnp.int32),
            pltpu.SMEM((P,), jnp.int32),
            pltpu.SMEM((P,), jnp.int32),
            pltpu.SMEM((P,), jnp.int32),
            pltpu.SemaphoreType.DMA((_NBUF, 3)),
        ],
    )
    fn = pl.pallas_call(
        functools.partial(_moe_body, top_k=K, n_experts=E),
        grid_spec=grid_spec,
        out_shape=jax.ShapeDtypeStruct((B, H), jnp.float32),
    )
    return fn(eids, ew, x, w1_stacked, w3_stacked, w2_stacked)


# speedup vs baseline: 1.0103x; 1.0009x over previous
import functools

import jax
import jax.numpy as jnp
from jax.experimental import pallas as pl
from jax.experimental.pallas import tpu as pltpu

_NBUF = 3


def _moe_body(eids_ref, ew_ref, x_ref, w1_any, w3_any, w2_any, out_ref,
              w1b, w3b, w2b, cnt, base, uexp, starts, scnt, srows, sems,
              *, top_k, n_experts):
    P = eids_ref.shape[0]
    out_ref[...] = jnp.zeros_like(out_ref)
    zero = jnp.int32(0)

    def clear_body(e, c):
        cnt[e] = zero
        return c
    jax.lax.fori_loop(0, n_experts, clear_body, zero)

    def count_body(i, c):
        cnt[eids_ref[i]] += 1
        return c
    jax.lax.fori_loop(0, P, count_body, zero)

    def seg_scan(e, carry):
        d, pos = carry
        c = cnt[e]

        @pl.when(c > 0)
        def _():
            uexp[d] = e
            starts[d] = pos
            scnt[d] = c
        base[e] = pos
        return jnp.where(c > 0, d + 1, d), pos + c

    d, _ = jax.lax.fori_loop(0, n_experts, seg_scan, (zero, zero))

    def scatter_body(i, c):
        e = eids_ref[i]
        b = base[e]
        srows[b] = i
        base[e] = b + 1
        return c
    jax.lax.fori_loop(0, P, scatter_body, zero)

    def _copies(j, slot):
        e = uexp[j]
        h2 = w1_any.shape[1] // 2
        i2 = w2_any.shape[1] // 2
        return (
            pltpu.make_async_copy(w1_any.at[e, pl.ds(0, h2)],
                                  w1b.at[slot, pl.ds(0, h2)], sems.at[slot, 0]),
            pltpu.make_async_copy(w1_any.at[e, pl.ds(h2, h2)],
                                  w1b.at[slot, pl.ds(h2, h2)], sems.at[slot, 1]),
            pltpu.make_async_copy(w3_any.at[e, pl.ds(0, h2)],
                                  w3b.at[slot, pl.ds(0, h2)], sems.at[slot, 2]),
            pltpu.make_async_copy(w3_any.at[e, pl.ds(h2, h2)],
                                  w3b.at[slot, pl.ds(h2, h2)], sems.at[slot, 3]),
            pltpu.make_async_copy(w2_any.at[e, pl.ds(0, i2)],
                                  w2b.at[slot, pl.ds(0, i2)], sems.at[slot, 4]),
            pltpu.make_async_copy(w2_any.at[e, pl.ds(i2, i2)],
                                  w2b.at[slot, pl.ds(i2, i2)], sems.at[slot, 5]),
        )

    for jj in range(_NBUF):
        @pl.when(jj < d)
        def _start():
            for c in _copies(jj, jj):
                c.start()

    def seg_body(j, carry):
        slot = jax.lax.rem(j, _NBUF)
        for c in _copies(j, slot):
            c.wait()

        start = starts[j]

        def row_body(r, carry2):
            p = srows[r]
            t = p // top_k
            xrow = x_ref[pl.ds(t, 1), :]
            g = jnp.dot(xrow, w1b[slot], preferred_element_type=jnp.float32)
            u = jnp.dot(xrow, w3b[slot], preferred_element_type=jnp.float32)
            h = (g * jax.nn.sigmoid(g)) * u
            o = jnp.dot(h, w2b[slot], preferred_element_type=jnp.float32)
            out_ref[pl.ds(t, 1), :] += ew_ref[p] * o
            return carry2

        jax.lax.fori_loop(start, start + scnt[j], row_body, zero)

        @pl.when(j + _NBUF < d)
        def _next():
            for c in _copies(j + _NBUF, slot):
                c.start()
        return carry

    jax.lax.fori_loop(0, d, seg_body, zero)


def kernel(x, expert_indices, expert_weights, w1_stacked, w2_stacked, w3_stacked):
    B, H = x.shape
    K = expert_indices.shape[1]
    E, _, I = w1_stacked.shape
    P = B * K

    eids = expert_indices.reshape(P).astype(jnp.int32)
    ew = expert_weights.reshape(P)

    grid_spec = pltpu.PrefetchScalarGridSpec(
        num_scalar_prefetch=2,
        grid=(1,),
        in_specs=[
            pl.BlockSpec((B, H), lambda i, *_: (0, 0)),
            pl.BlockSpec(memory_space=pl.ANY),
            pl.BlockSpec(memory_space=pl.ANY),
            pl.BlockSpec(memory_space=pl.ANY),
        ],
        out_specs=pl.BlockSpec((B, H), lambda i, *_: (0, 0)),
        scratch_shapes=[
            pltpu.VMEM((_NBUF, H, I), jnp.float32),
            pltpu.VMEM((_NBUF, H, I), jnp.float32),
            pltpu.VMEM((_NBUF, I, H), jnp.float32),
            pltpu.SMEM((E,), jnp.int32),
            pltpu.SMEM((E,), jnp.int32),
            pltpu.SMEM((P,), jnp.int32),
            pltpu.SMEM((P,), jnp.int32),
            pltpu.SMEM((P,), jnp.int32),
            pltpu.SMEM((P,), jnp.int32),
            pltpu.SemaphoreType.DMA((_NBUF, 6)),
        ],
    )
    fn = pl.pallas_call(
        functools.partial(_moe_body, top_k=K, n_experts=E),
        grid_spec=grid_spec,
        out_shape=jax.ShapeDtypeStruct((B, H), jnp.float32),
    )
    return fn(eids, ew, x, w1_stacked, w3_stacked, w2_stacked)
